# Initial kernel scaffold; baseline (speedup 1.0000x reference)
import jax
import jax.numpy as jnp
from jax.experimental import pallas as pl


def kernel(feat, edge_index, y, mask, W):
    raise NotImplementedError("write your pallas kernel here")


# speedup vs baseline: 16.2527x; 16.2527x over previous
import functools

import jax
import jax.numpy as jnp
from jax import lax
from jax.experimental import pallas as pl
from jax.experimental.pallas import tpu as pltpu
from jax.experimental.pallas import tpu_sc as plsc

N = 10000
E = 320000
F = 128
C = 40
ITERS = 2
M = 5000

D = 48
NP = 10240
NTILES = 16
NW = 32
RPT = NP // NTILES
K = 128
WCH0 = 82
WCH1 = 76
ZB = 160
TOTCH = NTILES * (WCH0 + WCH1)
EP = TOTCH * K
MCH = (M + NTILES * K - 1) // (NTILES * K)
MP = NTILES * MCH * K

_mesh = plsc.VectorSubcoreMesh(core_axis_name="c", subcore_axis_name="s")


def _zero_block(zb_v, width):
    z16 = jnp.zeros((16,), jnp.float32)

    def row(i, carry):
        for j in range(width // 16):
            zb_v[i, pl.ds(j * 16, 16)] = z16
        return carry

    lax.fori_loop(0, ZB, row, 0)


@functools.partial(
    pl.kernel,
    out_type=jax.ShapeDtypeStruct((N, D), jnp.float32),
    mesh=_mesh,
    compiler_params=pltpu.CompilerParams(use_tc_tiling_on_sc=False),
    scratch_types=[
        pltpu.VMEM((MCH, K), jnp.int32),
        pltpu.VMEM((MCH, K), jnp.int32),
        pltpu.VMEM((K, D), jnp.float32),
        pltpu.VMEM((K, D), jnp.float32),
        pltpu.VMEM_SHARED((N + 16, D), jnp.float32),
        pltpu.SemaphoreType.DMA,
    ],
)
def _s0_kernel(ty_hbm, gmask_hbm, smask_hbm, z48_hbm, out_hbm, gi_v, si_v,
               rows_v, zrows_v, s_sh, sem):
    cid = lax.axis_index("c")
    sid = lax.axis_index("s")
    rpt = N // NTILES

    @pl.when(cid == 0)
    def _fill():
        pltpu.sync_copy(z48_hbm.at[pl.ds(sid * rpt, rpt)],
                        s_sh.at[pl.ds(sid * rpt, rpt)])

    plsc.subcore_barrier()

    @pl.when(cid == 0)
    def _scatter():
        pltpu.sync_copy(gmask_hbm.at[sid], gi_v)
        pltpu.sync_copy(smask_hbm.at[sid], si_v)
        for j in range(MCH):
            pltpu.async_copy(ty_hbm.at[gi_v.at[j]], rows_v, sem).wait()
            pltpu.async_copy(z48_hbm.at[gi_v.at[j]], zrows_v, sem).wait()

            def addrow(i, carry):
                for c in range(D // 16):
                    rows_v[i, pl.ds(c * 16, 16)] = (
                        rows_v[i, pl.ds(c * 16, 16)]
                        + zrows_v[i, pl.ds(c * 16, 16)])
                return carry

            lax.fori_loop(0, K, addrow, 0)
            pltpu.sync_copy(rows_v, s_sh.at[si_v.at[j]])

    plsc.subcore_barrier()

    @pl.when(cid == 0)
    def _dump():
        pltpu.sync_copy(s_sh.at[pl.ds(sid * rpt, rpt)],
                        out_hbm.at[pl.ds(sid * rpt, rpt)])


def _make_spmm(width):
    @functools.partial(
        pl.kernel,
        out_type=jax.ShapeDtypeStruct((2, NP, width), jnp.float32),
        mesh=_mesh,
        compiler_params=pltpu.CompilerParams(use_tc_tiling_on_sc=False),
        scratch_types=[
            pltpu.VMEM((ZB, width), jnp.float32),
            pltpu.VMEM((WCH0, K), jnp.int32),
            pltpu.VMEM((WCH0, K), jnp.int32),
            pltpu.VMEM((K, width), jnp.float32),
            pltpu.VMEM((K, width), jnp.float32),
            pltpu.VMEM_SHARED((NP, width), jnp.float32),
            pltpu.VMEM_SHARED((N, width), jnp.float32),
            pltpu.SemaphoreType.DMA,
            pltpu.SemaphoreType.DMA,
        ],
    )
    def spmm(s_hbm, src_hbm, dst_hbm, out_hbm, zb_v, si_v, di_v, rows_v,
             rows_b, acc_sh, s_sh, sem, sem_b):
        cid = lax.axis_index("c")
        sid = lax.axis_index("s")

        pltpu.sync_copy(s_hbm.at[pl.ds(sid * (N // NTILES), N // NTILES)],
                        s_sh.at[pl.ds(sid * (N // NTILES), N // NTILES)])

        _zero_block(zb_v, width)
        base = sid * RPT

        def zfill(k, carry):
            pltpu.sync_copy(zb_v, acc_sh.at[pl.ds(base + k * ZB, ZB)])
            return carry

        lax.fori_loop(0, RPT // ZB, zfill, 0)

        def chunk(k, carry):
            j0 = 2 * k
            da = pltpu.async_copy(s_sh.at[si_v.at[j0]], rows_v, sem)
            db = pltpu.async_copy(s_sh.at[si_v.at[j0 + 1]], rows_b, sem_b)
            da.wait()
            pltpu.sync_copy(rows_v, acc_sh.at[di_v.at[j0]], add=True)
            db.wait()
            pltpu.sync_copy(rows_b, acc_sh.at[di_v.at[j0 + 1]], add=True)
            return carry

        @pl.when(cid == 0)
        def _core0():
            start = sid * WCH0
            pltpu.sync_copy(src_hbm.at[pl.ds(start, WCH0)],
                            si_v.at[pl.ds(0, WCH0)])
            pltpu.sync_copy(dst_hbm.at[pl.ds(start, WCH0)],
                            di_v.at[pl.ds(0, WCH0)])
            plsc.subcore_barrier()
            lax.fori_loop(0, WCH0 // 2, chunk, 0)

        @pl.when(cid == 1)
        def _core1():
            start = NTILES * WCH0 + sid * WCH1
            pltpu.sync_copy(src_hbm.at[pl.ds(start, WCH1)],
                            si_v.at[pl.ds(0, WCH1)])
            pltpu.sync_copy(dst_hbm.at[pl.ds(start, WCH1)],
                            di_v.at[pl.ds(0, WCH1)])
            plsc.subcore_barrier()
            lax.fori_loop(0, WCH1 // 2, chunk, 0)

        plsc.subcore_barrier()

        pltpu.sync_copy(acc_sh.at[pl.ds(base, RPT)],
                        out_hbm.at[cid, pl.ds(base, RPT)])

    return spmm


_spmm48 = _make_spmm(D)
_spmm40 = _make_spmm(C)


_RB = 2000


def _prep_body(feat_ref, w_ref, y_ref, z48_ref, ty_ref):
    w1 = w_ref[:F, :]
    z = jnp.dot(feat_ref[...], w1, preferred_element_type=jnp.float32)
    ones = jnp.ones((_RB, 1), jnp.float32)
    zeros = jnp.zeros((_RB, D - C - 1), jnp.float32)
    z48_ref[...] = jnp.concatenate([z, ones, zeros], axis=1)
    yv = y_ref[...]
    ohy = (yv == lax.broadcasted_iota(jnp.int32, (_RB, C), 1)).astype(
        jnp.float32)
    w2 = w_ref[F:, :]
    ty = jnp.dot(ohy, w2, preferred_element_type=jnp.float32)
    zc = jnp.zeros((_RB, 1), jnp.float32)
    ty_ref[...] = jnp.concatenate(
        [ty, zc, ones, jnp.zeros((_RB, D - C - 2), jnp.float32)], axis=1)


def _recycle(pred, s0_ref, z48_ref, w_ref):
    mx = jnp.max(pred, axis=1, keepdims=True)
    ex = jnp.exp(pred - mx)
    soft = ex / jnp.sum(ex, axis=1, keepdims=True)
    w2 = w_ref[F:, :]
    sw = jnp.dot(soft, w2, preferred_element_type=jnp.float32)
    s0 = s0_ref[...]
    return jnp.where(s0[:, C + 1:C + 2] > 0.5, s0[:, :C],
                     z48_ref[:, :C] + sw)


def _update1_body(p0_ref, p1_ref, z48_ref, lab_ref, w_ref, s_ref, deg_ref):
    agg = p0_ref[...] + p1_ref[...]
    deg = jnp.maximum(agg[:, C:C + 1], 1.0)
    deg_ref[...] = deg
    s_ref[...] = _recycle(agg[:, :C] / deg, lab_ref, z48_ref, w_ref)


def _update2_body(p0_ref, p1_ref, deg_ref, z48_ref, lab_ref, w_ref, s_ref):
    agg = p0_ref[...] + p1_ref[...]
    s_ref[...] = _recycle(agg / deg_ref[...], lab_ref, z48_ref, w_ref)


def _final_body(p0_ref, p1_ref, deg_ref, out_ref):
    out_ref[...] = (p0_ref[...] + p1_ref[...]) / deg_ref[...]


def _rows(width):
    return pl.BlockSpec((_RB, width), lambda i: (i, 0))


def _whole_w():
    return pl.BlockSpec((F + C, C), lambda i: (0, 0))


_GRID = (N // _RB,)

_prep_call = pl.pallas_call(
    _prep_body,
    grid=_GRID,
    in_specs=[_rows(F), _whole_w(), pl.BlockSpec((_RB, 1), lambda i: (i, 0))],
    out_specs=[_rows(D), _rows(D)],
    out_shape=[
        jax.ShapeDtypeStruct((N, D), jnp.float32),
        jax.ShapeDtypeStruct((N, D), jnp.float32),
    ],
)

_update1_call = pl.pallas_call(
    _update1_body,
    grid=_GRID,
    in_specs=[_rows(D), _rows(D), _rows(D), _rows(D), _whole_w()],
    out_specs=[_rows(C), _rows(1)],
    out_shape=[
        jax.ShapeDtypeStruct((N, C), jnp.float32),
        jax.ShapeDtypeStruct((N, 1), jnp.float32),
    ],
)

_update2_call = pl.pallas_call(
    _update2_body,
    grid=_GRID,
    in_specs=[_rows(C), _rows(C), _rows(1), _rows(D), _rows(D), _whole_w()],
    out_specs=_rows(C),
    out_shape=jax.ShapeDtypeStruct((N, C), jnp.float32),
)

_final_call = pl.pallas_call(
    _final_body,
    grid=_GRID,
    in_specs=[_rows(C), _rows(C), _rows(1)],
    out_specs=_rows(C),
    out_shape=jax.ShapeDtypeStruct((N, C), jnp.float32),
)


def kernel(feat, edge_index, y, mask, W):
    src = edge_index[0].astype(jnp.int32)
    dst = edge_index[1].astype(jnp.int32)
    epad = EP - E
    srcp = jnp.concatenate([src, jnp.zeros((epad,), jnp.int32)]).reshape(
        TOTCH, K)
    dstp = jnp.concatenate([dst, jnp.full((epad,), N, jnp.int32)]).reshape(
        TOTCH, K)
    mask = mask.astype(jnp.int32)
    mpad = MP - M
    gmask = jnp.concatenate([mask, jnp.zeros((mpad,), jnp.int32)]).reshape(
        NTILES, MCH, K)
    smask = jnp.concatenate([mask, jnp.full((mpad,), N, jnp.int32)]).reshape(
        NTILES, MCH, K)

    z48, ty = _prep_call(feat, W, y.reshape(N, 1))
    s0 = _s0_kernel(ty, gmask, smask, z48)
    parts = _spmm48(s0, srcp, dstp)
    s, deg = _update1_call(parts[0, :N], parts[1, :N], z48, s0, W)
    parts = _spmm40(s, srcp, dstp)
    s = _update2_call(parts[0, :N], parts[1, :N], deg, z48, s0, W)
    parts = _spmm40(s, srcp, dstp)
    return _final_call(parts[0, :N], parts[1, :N], deg)
